# baseline (speedup 1.0000x reference)
import jax
import jax.numpy as jnp
from jax import lax
from jax.experimental import pallas as pl
from jax.experimental.pallas import tpu as pltpu

S = 14
L_COORD = 5.0
L_NOOBJ = 0.5
N_CLS = 20
N_BATCH = 256
_NCELL = N_BATCH * S * S
_RB = 1
_GRID = S // _RB


def _loss_kernel(pred_ref, tbox_ref, tcls_ref, mask_ref, out_ref, acc_ref):
    step = pl.program_id(0)
    mask = mask_ref[...].astype(jnp.float32)
    no_mask = 1.0 - mask

    logits = pred_ref[:, :, 10:30, :]
    m = jnp.max(logits, axis=2, keepdims=True)
    se = jnp.sum(jnp.exp(logits - m), axis=2)
    tm = tcls_ref[:, 0, :, :]
    for c in range(1, N_CLS):
        tm = jnp.maximum(tm, tcls_ref[:, c, :, :])
    idx = jnp.full(tm.shape, N_CLS, jnp.int32)
    for c in range(N_CLS - 1, -1, -1):
        idx = jnp.where(tcls_ref[:, c, :, :] == tm, c, idx)
    iot = lax.broadcasted_iota(jnp.int32, logits.shape, 2)
    sel = jnp.sum(jnp.where(iot == idx[:, :, None, :], logits, 0.0), axis=2)
    ce = m[:, :, 0, :] + jnp.log(se) - sel

    conf0 = pred_ref[:, :, 4, :]
    conf1 = pred_ref[:, :, 9, :]

    inv_s = 1.0 / S
    tbx = tbox_ref[:, :, 0, :]
    tby = tbox_ref[:, :, 1, :]
    tbw = tbox_ref[:, :, 2, :]
    tbh = tbox_ref[:, :, 3, :]
    tx1 = tbx * inv_s - 0.5 * tbw
    ty1 = tby * inv_s - 0.5 * tbh
    tx2 = tbx * inv_s + 0.5 * tbw
    ty2 = tby * inv_s + 0.5 * tbh
    t_area = (tx2 - tx1) * (ty2 - ty1)

    def box(o):
        px = pred_ref[:, :, o, :]
        py = pred_ref[:, :, o + 1, :]
        pw = pred_ref[:, :, o + 2, :]
        ph = pred_ref[:, :, o + 3, :]
        x1 = px * inv_s - 0.5 * pw
        y1 = py * inv_s - 0.5 * ph
        x2 = px * inv_s + 0.5 * pw
        y2 = py * inv_s + 0.5 * ph
        ix = jnp.maximum(jnp.minimum(x2, tx2) - jnp.maximum(x1, tx1), 0.0)
        iy = jnp.maximum(jnp.minimum(y2, ty2) - jnp.maximum(y1, ty1), 0.0)
        inter = ix * iy
        union = (x2 - x1) * (y2 - y1) + t_area - inter
        iou = inter / jnp.maximum(union, 1e-9)
        return (x1, y1, x2, y2), iou

    (b0, iou0), (b1, iou1) = box(0), box(5)
    upd = iou1 > iou0
    best_iou = jnp.where(upd, iou1, iou0)
    best_conf = jnp.where(upd, conf1, conf0)

    reg = jnp.zeros_like(mask)
    for p0, p1, tc in zip(b0, b1, (tx1, ty1, tx2, ty2)):
        d = jnp.where(upd, p1, p0) - tc
        reg = reg + d * d

    dcf = best_conf - best_iou

    p_mask = jnp.sum(mask)
    p_ce = jnp.sum(mask * ce)
    p_noobj = jnp.sum(no_mask * (conf0 * conf0 + conf1 * conf1))
    p_reg = jnp.sum(mask * reg)
    p_contain = jnp.sum(mask * dcf * dcf)

    @pl.when(step == 0)
    def _init():
        acc_ref[0] = p_mask
        acc_ref[1] = p_ce
        acc_ref[2] = p_noobj
        acc_ref[3] = p_reg
        acc_ref[4] = p_contain

    @pl.when(step != 0)
    def _acc():
        acc_ref[0] += p_mask
        acc_ref[1] += p_ce
        acc_ref[2] += p_noobj
        acc_ref[3] += p_reg
        acc_ref[4] += p_contain

    @pl.when(step == _GRID - 1)
    def _fin():
        n_obj = jnp.maximum(acc_ref[0], 1.0)
        n_noobj = jnp.maximum(float(_NCELL) - acc_ref[0], 1.0)
        total = (1.0 / N_BATCH) * (L_COORD * acc_ref[3] + acc_ref[4]
                                   + L_NOOBJ * acc_ref[2] / n_noobj
                                   + acc_ref[1] / n_obj)
        out_ref[:, :] = jnp.broadcast_to(total, (1, 1))


def kernel(pred_tensor, target_boxes, target_cls, has_object_map):
    pred_t = jnp.transpose(pred_tensor, (1, 2, 3, 0))
    tbox_t = jnp.transpose(target_boxes, (1, 2, 3, 0))
    tcls_t = jnp.transpose(target_cls, (1, 3, 2, 0))
    mask_t = jnp.transpose(has_object_map, (1, 2, 0))

    out = pl.pallas_call(
        _loss_kernel,
        grid=(_GRID,),
        in_specs=[
            pl.BlockSpec((_RB, S, 30, N_BATCH), lambda i: (i, 0, 0, 0)),
            pl.BlockSpec((_RB, S, 4, N_BATCH), lambda i: (i, 0, 0, 0)),
            pl.BlockSpec((_RB, N_CLS, S, N_BATCH), lambda i: (i, 0, 0, 0)),
            pl.BlockSpec((_RB, S, N_BATCH), lambda i: (i, 0, 0)),
        ],
        out_specs=pl.BlockSpec((1, 1), lambda i: (0, 0)),
        out_shape=jax.ShapeDtypeStruct((1, 1), jnp.float32),
        scratch_shapes=[pltpu.SMEM((8,), jnp.float32)],
    )(pred_t, tbox_t, tcls_t, mask_t)
    return out[0, 0]


# speedup vs baseline: 1.2061x; 1.2061x over previous
import jax
import jax.numpy as jnp
from jax import lax
from jax.experimental import pallas as pl
from jax.experimental.pallas import tpu as pltpu

S = 14
L_COORD = 5.0
L_NOOBJ = 0.5
N_CLS = 20
N_BATCH = 256
_NCELL = N_BATCH * S * S
_RB = 2
_GRID = S // _RB


def _loss_kernel(pred_ref, tbox_ref, tcls_ref, mask_ref, out_ref, acc_ref):
    step = pl.program_id(0)
    mask = mask_ref[...].astype(jnp.float32)
    no_mask = 1.0 - mask

    logits = pred_ref[:, :, 10:30, :]
    se = jnp.sum(jnp.exp(logits), axis=2)
    tm = tcls_ref[:, 0, :, :]
    for c in range(1, N_CLS):
        tm = jnp.maximum(tm, tcls_ref[:, c, :, :])
    idx = jnp.full(tm.shape, N_CLS, jnp.int32)
    for c in range(N_CLS - 1, -1, -1):
        idx = jnp.where(tcls_ref[:, c, :, :] == tm, c, idx)
    iot = lax.broadcasted_iota(jnp.int32, logits.shape, 2)
    sel = jnp.sum(jnp.where(iot == idx[:, :, None, :], logits, 0.0), axis=2)
    ce = jnp.log(se) - sel

    conf0 = pred_ref[:, :, 4, :]
    conf1 = pred_ref[:, :, 9, :]

    inv_s = 1.0 / S
    tbx = tbox_ref[:, :, 0, :]
    tby = tbox_ref[:, :, 1, :]
    tbw = tbox_ref[:, :, 2, :]
    tbh = tbox_ref[:, :, 3, :]
    tx1 = tbx * inv_s - 0.5 * tbw
    ty1 = tby * inv_s - 0.5 * tbh
    tx2 = tbx * inv_s + 0.5 * tbw
    ty2 = tby * inv_s + 0.5 * tbh
    t_area = (tx2 - tx1) * (ty2 - ty1)

    def box(o):
        px = pred_ref[:, :, o, :]
        py = pred_ref[:, :, o + 1, :]
        pw = pred_ref[:, :, o + 2, :]
        ph = pred_ref[:, :, o + 3, :]
        x1 = px * inv_s - 0.5 * pw
        y1 = py * inv_s - 0.5 * ph
        x2 = px * inv_s + 0.5 * pw
        y2 = py * inv_s + 0.5 * ph
        ix = jnp.maximum(jnp.minimum(x2, tx2) - jnp.maximum(x1, tx1), 0.0)
        iy = jnp.maximum(jnp.minimum(y2, ty2) - jnp.maximum(y1, ty1), 0.0)
        inter = ix * iy
        union = (x2 - x1) * (y2 - y1) + t_area - inter
        iou = inter / jnp.maximum(union, 1e-9)
        return (x1, y1, x2, y2), iou

    (b0, iou0), (b1, iou1) = box(0), box(5)
    upd = iou1 > iou0
    best_iou = jnp.where(upd, iou1, iou0)
    best_conf = jnp.where(upd, conf1, conf0)

    reg = jnp.zeros_like(mask)
    for p0, p1, tc in zip(b0, b1, (tx1, ty1, tx2, ty2)):
        d = jnp.where(upd, p1, p0) - tc
        reg = reg + d * d

    dcf = best_conf - best_iou

    p_mask = jnp.sum(mask)
    p_ce = jnp.sum(mask * ce)
    p_noobj = jnp.sum(no_mask * (conf0 * conf0 + conf1 * conf1))
    p_reg = jnp.sum(mask * reg)
    p_contain = jnp.sum(mask * dcf * dcf)

    @pl.when(step == 0)
    def _init():
        acc_ref[0] = p_mask
        acc_ref[1] = p_ce
        acc_ref[2] = p_noobj
        acc_ref[3] = p_reg
        acc_ref[4] = p_contain

    @pl.when(step != 0)
    def _acc():
        acc_ref[0] += p_mask
        acc_ref[1] += p_ce
        acc_ref[2] += p_noobj
        acc_ref[3] += p_reg
        acc_ref[4] += p_contain

    @pl.when(step == _GRID - 1)
    def _fin():
        n_obj = jnp.maximum(acc_ref[0], 1.0)
        n_noobj = jnp.maximum(float(_NCELL) - acc_ref[0], 1.0)
        total = (1.0 / N_BATCH) * (L_COORD * acc_ref[3] + acc_ref[4]
                                   + L_NOOBJ * acc_ref[2] / n_noobj
                                   + acc_ref[1] / n_obj)
        out_ref[:, :] = jnp.broadcast_to(total, (1, 1))


def kernel(pred_tensor, target_boxes, target_cls, has_object_map):
    pred_t = jnp.transpose(pred_tensor, (1, 2, 3, 0))
    tbox_t = jnp.transpose(target_boxes, (1, 2, 3, 0))
    tcls_t = jnp.transpose(target_cls, (1, 3, 2, 0))
    mask_t = jnp.transpose(has_object_map, (1, 2, 0))

    out = pl.pallas_call(
        _loss_kernel,
        grid=(_GRID,),
        in_specs=[
            pl.BlockSpec((_RB, S, 30, N_BATCH), lambda i: (i, 0, 0, 0)),
            pl.BlockSpec((_RB, S, 4, N_BATCH), lambda i: (i, 0, 0, 0)),
            pl.BlockSpec((_RB, N_CLS, S, N_BATCH), lambda i: (i, 0, 0, 0)),
            pl.BlockSpec((_RB, S, N_BATCH), lambda i: (i, 0, 0)),
        ],
        out_specs=pl.BlockSpec((1, 1), lambda i: (0, 0)),
        out_shape=jax.ShapeDtypeStruct((1, 1), jnp.float32),
        scratch_shapes=[pltpu.SMEM((8,), jnp.float32)],
    )(pred_t, tbox_t, tcls_t, mask_t)
    return out[0, 0]


# speedup vs baseline: 1.2282x; 1.0183x over previous
import jax
import jax.numpy as jnp
from jax import lax
from jax.experimental import pallas as pl
from jax.experimental.pallas import tpu as pltpu

S = 14
L_COORD = 5.0
L_NOOBJ = 0.5
N_CLS = 20
N_BATCH = 256
_NCELL = N_BATCH * S * S
_RB = 2
_GRID = S // _RB


def _loss_kernel(pred_ref, tbox_ref, tcls_ref, mask_ref, out_ref, acc_ref, accv_ref):
    step = pl.program_id(0)
    mask = mask_ref[...].astype(jnp.float32)
    no_mask = 1.0 - mask

    logits = pred_ref[:, :, 10:30, :]
    se = jnp.sum(jnp.exp(logits), axis=2)
    tm = tcls_ref[:, 0, :, :]
    for c in range(1, N_CLS):
        tm = jnp.maximum(tm, tcls_ref[:, c, :, :])
    idx = jnp.full(tm.shape, N_CLS, jnp.int32)
    for c in range(N_CLS - 1, -1, -1):
        idx = jnp.where(tcls_ref[:, c, :, :] == tm, c, idx)
    iot = lax.broadcasted_iota(jnp.int32, logits.shape, 2)
    sel = jnp.sum(jnp.where(iot == idx[:, :, None, :], logits, 0.0), axis=2)
    ce = jnp.log(se) - sel

    conf0 = pred_ref[:, :, 4, :]
    conf1 = pred_ref[:, :, 9, :]

    inv_s = 1.0 / S
    tbx = tbox_ref[:, :, 0, :]
    tby = tbox_ref[:, :, 1, :]
    tbw = tbox_ref[:, :, 2, :]
    tbh = tbox_ref[:, :, 3, :]
    tx1 = tbx * inv_s - 0.5 * tbw
    ty1 = tby * inv_s - 0.5 * tbh
    tx2 = tbx * inv_s + 0.5 * tbw
    ty2 = tby * inv_s + 0.5 * tbh
    t_area = (tx2 - tx1) * (ty2 - ty1)

    def box(o):
        px = pred_ref[:, :, o, :]
        py = pred_ref[:, :, o + 1, :]
        pw = pred_ref[:, :, o + 2, :]
        ph = pred_ref[:, :, o + 3, :]
        x1 = px * inv_s - 0.5 * pw
        y1 = py * inv_s - 0.5 * ph
        x2 = px * inv_s + 0.5 * pw
        y2 = py * inv_s + 0.5 * ph
        ix = jnp.maximum(jnp.minimum(x2, tx2) - jnp.maximum(x1, tx1), 0.0)
        iy = jnp.maximum(jnp.minimum(y2, ty2) - jnp.maximum(y1, ty1), 0.0)
        inter = ix * iy
        union = (x2 - x1) * (y2 - y1) + t_area - inter
        iou = inter / jnp.maximum(union, 1e-9)
        return (x1, y1, x2, y2), iou

    (b0, iou0), (b1, iou1) = box(0), box(5)
    upd = iou1 > iou0
    best_iou = jnp.where(upd, iou1, iou0)
    best_conf = jnp.where(upd, conf1, conf0)

    reg = jnp.zeros_like(mask)
    for p0, p1, tc in zip(b0, b1, (tx1, ty1, tx2, ty2)):
        d = jnp.where(upd, p1, p0) - tc
        reg = reg + d * d

    dcf = best_conf - best_iou

    p_mask = mask
    p_ce = mask * ce
    p_noobj = no_mask * (conf0 * conf0 + conf1 * conf1)
    p_reg = mask * reg
    p_contain = mask * dcf * dcf
    planes = (p_mask, p_ce, p_noobj, p_reg, p_contain)

    @pl.when(step == 0)
    def _init():
        for k, p in enumerate(planes):
            accv_ref[k] = p

    @pl.when(step != 0)
    def _acc():
        for k, p in enumerate(planes):
            accv_ref[k] += p

    @pl.when(step == _GRID - 1)
    def _fin():
        s_mask = jnp.sum(accv_ref[0])
        s_ce = jnp.sum(accv_ref[1])
        s_noobj = jnp.sum(accv_ref[2])
        s_reg = jnp.sum(accv_ref[3])
        s_contain = jnp.sum(accv_ref[4])
        n_obj = jnp.maximum(s_mask, 1.0)
        n_noobj = jnp.maximum(float(_NCELL) - s_mask, 1.0)
        total = (1.0 / N_BATCH) * (L_COORD * s_reg + s_contain
                                   + L_NOOBJ * s_noobj / n_noobj
                                   + s_ce / n_obj)
        out_ref[:, :] = jnp.broadcast_to(total, (1, 1))


def kernel(pred_tensor, target_boxes, target_cls, has_object_map):
    pred_t = jnp.transpose(pred_tensor, (1, 2, 3, 0))
    tbox_t = jnp.transpose(target_boxes, (1, 2, 3, 0))
    tcls_t = jnp.transpose(target_cls, (1, 3, 2, 0))
    mask_t = jnp.transpose(has_object_map, (1, 2, 0))

    out = pl.pallas_call(
        _loss_kernel,
        grid=(_GRID,),
        in_specs=[
            pl.BlockSpec((_RB, S, 30, N_BATCH), lambda i: (i, 0, 0, 0)),
            pl.BlockSpec((_RB, S, 4, N_BATCH), lambda i: (i, 0, 0, 0)),
            pl.BlockSpec((_RB, N_CLS, S, N_BATCH), lambda i: (i, 0, 0, 0)),
            pl.BlockSpec((_RB, S, N_BATCH), lambda i: (i, 0, 0)),
        ],
        out_specs=pl.BlockSpec((1, 1), lambda i: (0, 0)),
        out_shape=jax.ShapeDtypeStruct((1, 1), jnp.float32),
        scratch_shapes=[pltpu.SMEM((8,), jnp.float32),
                        pltpu.VMEM((5, _RB, S, N_BATCH), jnp.float32)],
    )(pred_t, tbox_t, tcls_t, mask_t)
    return out[0, 0]
